# baseline (speedup 1.0000x reference)
import functools

import jax
import jax.numpy as jnp
from jax import lax
from jax.experimental import pallas as pl
from jax.experimental.pallas import tpu as pltpu
from jax.experimental.pallas import tpu_sc as plsc

N = 10000
E = 320000
CH = 128
NTILES = 16
NPT = N // NTILES
K_PT = 160
E_PAD = NTILES * K_PT * CH
BLK = 1000
GRID = N // BLK


def _prep_body(h_ref, wa_ref, ba_ref, hwa_ref, hwb_ref, ed_ref):
    h = h_ref[...]
    d = h.shape[1]
    w_half = hwa_ref.shape[1]
    a = jnp.dot(h, wa_ref[...], preferred_element_type=jnp.float32)
    a_dst = a[:, 0:1]
    a_src = a[:, 1:2]
    w = jnp.exp(a_src)
    ed_ref[...] = jnp.exp(a_dst + ba_ref[0, 0])
    hw = h * w
    pad = jnp.concatenate(
        [hw, w, jnp.zeros((h.shape[0], 2 * w_half - d - 1), jnp.float32)],
        axis=1)
    hwa_ref[...] = pad[:, :w_half]
    hwb_ref[...] = pad[:, w_half:]


def _make_prep(d, w_half):
    return pl.pallas_call(
        _prep_body,
        grid=(GRID,),
        in_specs=[
            pl.BlockSpec((BLK, d), lambda i: (i, 0)),
            pl.BlockSpec((d, 2), lambda i: (0, 0)),
            pl.BlockSpec((1, 1), lambda i: (0, 0)),
        ],
        out_specs=[
            pl.BlockSpec((BLK, w_half), lambda i: (i, 0)),
            pl.BlockSpec((BLK, w_half), lambda i: (i, 0)),
            pl.BlockSpec((BLK, 1), lambda i: (i, 0)),
        ],
        out_shape=[
            jax.ShapeDtypeStruct((N, w_half), jnp.float32),
            jax.ShapeDtypeStruct((N, w_half), jnp.float32),
            jax.ShapeDtypeStruct((N, 1), jnp.float32),
        ],
    )


def _make_sc(w_half):
    mesh = plsc.VectorSubcoreMesh(core_axis_name="c", subcore_axis_name="s")

    @functools.partial(
        pl.kernel,
        mesh=mesh,
        compiler_params=pltpu.CompilerParams(use_tc_tiling_on_sc=False),
        out_type=[
            jax.ShapeDtypeStruct((N, w_half), jnp.float32),
            jax.ShapeDtypeStruct((N, w_half), jnp.float32),
        ],
        scratch_types=[
            pltpu.VMEM((4, CH), jnp.int32),
            pltpu.VMEM((4, CH), jnp.int32),
            pltpu.VMEM((CH, w_half), jnp.float32),
            pltpu.VMEM((CH, w_half), jnp.float32),
            pltpu.VMEM_SHARED((N + 8, w_half), jnp.float32),
            pltpu.SemaphoreType.DMA,
            pltpu.SemaphoreType.DMA,
            pltpu.SemaphoreType.DMA,
            pltpu.SemaphoreType.DMA,
            pltpu.SemaphoreType.DMA,
            pltpu.SemaphoreType.DMA,
            pltpu.SemaphoreType.DMA,
            pltpu.SemaphoreType.DMA,
        ],
    )
    def sc_kernel(hwa, hwb, src_r, dst_r, zblk, zblk8, ua, ub,
                  sidx, didx, rows0, rows1, ush,
                  si0, si1, si2, si3, sg0, sg1, ss0, ss1):
        c = lax.axis_index("c")
        s = lax.axis_index("s")
        pltpu.sync_copy(zblk, ush.at[pl.ds(s * NPT, NPT)])

        @pl.when(s == NTILES - 1)
        def _():
            pltpu.sync_copy(zblk8, ush.at[pl.ds(N, 8)])

        plsc.subcore_barrier()

        rows = (rows0, rows1)
        si = (si0, si1, si2, si3)
        sg = (sg0, sg1)
        ss = (ss0, ss1)

        def fetch_idx(k, u):
            pltpu.async_copy(src_r.at[s, k], sidx.at[u], si[u])
            pltpu.async_copy(dst_r.at[s, k], didx.at[u], si[u])

        def wait_idx(k, u):
            pltpu.make_async_copy(src_r.at[s, k], sidx.at[u], si[u]).wait()
            pltpu.make_async_copy(dst_r.at[s, k], didx.at[u], si[u]).wait()

        def run(hw_ref):
            fetch_idx(0, 0)
            fetch_idx(1, 1)
            wait_idx(0, 0)
            pltpu.async_copy(hw_ref.at[sidx.at[0]], rows[0], sg[0])

            def body_fn(j, carry):
                for r in range(4):
                    k = 4 * j + r
                    b = r % 2
                    nb = 1 - b
                    pltpu.make_async_copy(
                        hw_ref.at[sidx.at[r]], rows[b], sg[b]).wait()

                    @pl.when(k >= 1)
                    def _():
                        pltpu.make_async_copy(
                            rows[nb], ush.at[didx.at[(r + 3) % 4]],
                            ss[nb]).wait()

                    @pl.when(k + 2 < K_PT)
                    def _():
                        fetch_idx(k + 2, (r + 2) % 4)

                    @pl.when(k + 1 < K_PT)
                    def _():
                        wait_idx(k + 1, (r + 1) % 4)
                        pltpu.async_copy(
                            hw_ref.at[sidx.at[(r + 1) % 4]], rows[nb], sg[nb])

                    pltpu.async_copy(
                        rows[b], ush.at[didx.at[r]], ss[b], add=True)
                return carry

            lax.fori_loop(0, K_PT // 4, body_fn, 0)
            pltpu.make_async_copy(
                rows[1], ush.at[didx.at[3]], ss[1]).wait()

        @pl.when(c == 0)
        def _():
            run(hwa)

        @pl.when(c == 1)
        def _():
            run(hwb)

        plsc.subcore_barrier()

        @pl.when(c == 0)
        def _():
            pltpu.sync_copy(ush.at[pl.ds(s * NPT, NPT)],
                            ua.at[pl.ds(s * NPT, NPT)])

        @pl.when(c == 1)
        def _():
            pltpu.sync_copy(ush.at[pl.ds(s * NPT, NPT)],
                            ub.at[pl.ds(s * NPT, NPT)])

    return sc_kernel


def _combine_body(ua_ref, ub_ref, ed_ref, h_ref, w1t_ref, w2t_ref,
                  bs_ref, g_ref, be_ref, out_ref):
    d = h_ref.shape[1]
    u = jnp.concatenate([ua_ref[...], ub_ref[...]], axis=1)
    t = u[:, :d]
    s_sum = u[:, d:d + 1]
    ed = ed_ref[...]
    r = ed / jnp.maximum(ed * s_sum, 1e-8)
    agg = t * r
    z = (jnp.dot(h_ref[...], w1t_ref[...], preferred_element_type=jnp.float32)
         + jnp.dot(agg, w2t_ref[...], preferred_element_type=jnp.float32)
         + bs_ref[...])
    mu = jnp.mean(z, axis=1, keepdims=True)
    var = jnp.mean((z - mu) ** 2, axis=1, keepdims=True)
    out_ref[...] = jnp.maximum(
        (z - mu) * lax.rsqrt(var + 1e-5) * g_ref[...] + be_ref[...], 0.0)


def _make_combine(d, w_half):
    return pl.pallas_call(
        _combine_body,
        grid=(GRID,),
        in_specs=[
            pl.BlockSpec((BLK, w_half), lambda i: (i, 0)),
            pl.BlockSpec((BLK, w_half), lambda i: (i, 0)),
            pl.BlockSpec((BLK, 1), lambda i: (i, 0)),
            pl.BlockSpec((BLK, d), lambda i: (i, 0)),
            pl.BlockSpec((d, 256), lambda i: (0, 0)),
            pl.BlockSpec((d, 256), lambda i: (0, 0)),
            pl.BlockSpec((1, 256), lambda i: (0, 0)),
            pl.BlockSpec((1, 256), lambda i: (0, 0)),
            pl.BlockSpec((1, 256), lambda i: (0, 0)),
        ],
        out_specs=pl.BlockSpec((BLK, 256), lambda i: (i, 0)),
        out_shape=jax.ShapeDtypeStruct((N, 256), jnp.float32),
    )


def _combine_final_body(ua_ref, ub_ref, ed_ref, h_ref, w1t_ref, w2t_ref,
                        bs_ref, g_ref, be_ref, x_ref, wskipt_ref,
                        bskip_ref, acc_ref):
    d = h_ref.shape[1]
    u = jnp.concatenate([ua_ref[...], ub_ref[...]], axis=1)
    t = u[:, :d]
    s_sum = u[:, d:d + 1]
    ed = ed_ref[...]
    r = ed / jnp.maximum(ed * s_sum, 1e-8)
    agg = t * r
    z = (jnp.dot(h_ref[...], w1t_ref[...], preferred_element_type=jnp.float32)
         + jnp.dot(agg, w2t_ref[...], preferred_element_type=jnp.float32)
         + bs_ref[...])
    mu = jnp.mean(z, axis=1, keepdims=True)
    var = jnp.mean((z - mu) ** 2, axis=1, keepdims=True)
    h3 = jnp.maximum(
        (z - mu) * lax.rsqrt(var + 1e-5) * g_ref[...] + be_ref[...], 0.0)
    xs = jnp.sum(x_ref[...], axis=0, keepdims=True)
    part = (jnp.sum(h3, axis=0, keepdims=True)
            + jnp.dot(xs, wskipt_ref[...], preferred_element_type=jnp.float32))

    @pl.when(pl.program_id(0) == 0)
    def _():
        acc_ref[...] = jnp.zeros_like(acc_ref)

    acc_ref[...] += part

    @pl.when(pl.program_id(0) == GRID - 1)
    def _():
        acc_ref[...] = acc_ref[...] * (1.0 / N) + bskip_ref[...]


def _make_combine_final(d, w_half, in_dim):
    return pl.pallas_call(
        _combine_final_body,
        grid=(GRID,),
        in_specs=[
            pl.BlockSpec((BLK, w_half), lambda i: (i, 0)),
            pl.BlockSpec((BLK, w_half), lambda i: (i, 0)),
            pl.BlockSpec((BLK, 1), lambda i: (i, 0)),
            pl.BlockSpec((BLK, d), lambda i: (i, 0)),
            pl.BlockSpec((d, 256), lambda i: (0, 0)),
            pl.BlockSpec((d, 256), lambda i: (0, 0)),
            pl.BlockSpec((1, 256), lambda i: (0, 0)),
            pl.BlockSpec((1, 256), lambda i: (0, 0)),
            pl.BlockSpec((1, 256), lambda i: (0, 0)),
            pl.BlockSpec((BLK, in_dim), lambda i: (i, 0)),
            pl.BlockSpec((in_dim, 256), lambda i: (0, 0)),
            pl.BlockSpec((1, 256), lambda i: (0, 0)),
        ],
        out_specs=pl.BlockSpec((1, 256), lambda i: (0, 0)),
        out_shape=jax.ShapeDtypeStruct((1, 256), jnp.float32),
    )


def kernel(x, edge_index, Ws0, bs0, g0, be0, Wa0, ba0, Ws1, bs1, g1, be1,
           Wa1, ba1, Ws2, bs2, g2, be2, Wa2, ba2, Wskip, bskip):
    pad = E_PAD - E
    src3 = jnp.concatenate(
        [edge_index[0], jnp.zeros((pad,), jnp.int32)]).reshape(
            NTILES, K_PT, CH)
    dst3 = jnp.concatenate(
        [edge_index[1], jnp.full((pad,), N, jnp.int32)]).reshape(
            NTILES, K_PT, CH)

    dims = (128, 256, 256)
    halves = (72, 136, 136)
    params = ((Ws0, bs0, g0, be0, Wa0, ba0),
              (Ws1, bs1, g1, be1, Wa1, ba1),
              (Ws2, bs2, g2, be2, Wa2, ba2))

    h = x
    for i in range(3):
        d = dims[i]
        w_half = halves[i]
        Ws, bs, g, be, Wa, ba = params[i]
        wa2 = jnp.stack([Wa[0, :d], Wa[0, d:]], axis=1)
        ba_arr = ba.reshape(1, 1)
        hwa, hwb, ed = _make_prep(d, w_half)(h, wa2, ba_arr)
        zblk = jnp.zeros((NPT, w_half), jnp.float32)
        zblk8 = jnp.zeros((8, w_half), jnp.float32)
        ua, ub = _make_sc(w_half)(hwa, hwb, src3, dst3, zblk, zblk8)
        w1t = Ws[:, :d].T
        w2t = Ws[:, d:].T
        if i < 2:
            h = _make_combine(d, w_half)(
                ua, ub, ed, h, w1t, w2t,
                bs.reshape(1, 256), g.reshape(1, 256), be.reshape(1, 256))
        else:
            acc = _make_combine_final(d, w_half, 128)(
                ua, ub, ed, h, w1t, w2t,
                bs.reshape(1, 256), g.reshape(1, 256), be.reshape(1, 256),
                x, Wskip.T, bskip.reshape(1, 256))
    return acc.reshape(256)


# speedup vs baseline: 1.0879x; 1.0879x over previous
import functools

import jax
import jax.numpy as jnp
from jax import lax
from jax.experimental import pallas as pl
from jax.experimental.pallas import tpu as pltpu
from jax.experimental.pallas import tpu_sc as plsc

N = 10000
E = 320000
CH = 128
NTILES = 16
NPT = N // NTILES
K_PT = 160
E_PAD = NTILES * K_PT * CH
BLK = 1000
GRID = N // BLK


def _prep_body(h_ref, wa_ref, ba_ref, hwa_ref, hwb_ref, ed_ref):
    h = h_ref[...]
    d = h.shape[1]
    w_half = hwa_ref.shape[1]
    a = jnp.dot(h, wa_ref[...], preferred_element_type=jnp.float32)
    a_dst = a[:, 0:1]
    a_src = a[:, 1:2]
    w = jnp.exp(a_src)
    ed_ref[...] = jnp.exp(a_dst + ba_ref[0, 0])
    hw = h * w
    pad = jnp.concatenate(
        [hw, w, jnp.zeros((h.shape[0], 2 * w_half - d - 1), jnp.float32)],
        axis=1)
    hwa_ref[...] = pad[:, :w_half]
    hwb_ref[...] = pad[:, w_half:]


def _make_prep(d, w_half):
    return pl.pallas_call(
        _prep_body,
        grid=(GRID,),
        in_specs=[
            pl.BlockSpec((BLK, d), lambda i: (i, 0)),
            pl.BlockSpec((d, 2), lambda i: (0, 0)),
            pl.BlockSpec((1, 1), lambda i: (0, 0)),
        ],
        out_specs=[
            pl.BlockSpec((BLK, w_half), lambda i: (i, 0)),
            pl.BlockSpec((BLK, w_half), lambda i: (i, 0)),
            pl.BlockSpec((BLK, 1), lambda i: (i, 0)),
        ],
        out_shape=[
            jax.ShapeDtypeStruct((N, w_half), jnp.float32),
            jax.ShapeDtypeStruct((N, w_half), jnp.float32),
            jax.ShapeDtypeStruct((N, 1), jnp.float32),
        ],
    )


def _make_sc(w_half):
    mesh = plsc.VectorSubcoreMesh(core_axis_name="c", subcore_axis_name="s")

    @functools.partial(
        pl.kernel,
        mesh=mesh,
        compiler_params=pltpu.CompilerParams(use_tc_tiling_on_sc=False),
        out_type=[
            jax.ShapeDtypeStruct((N, w_half), jnp.float32),
            jax.ShapeDtypeStruct((N, w_half), jnp.float32),
        ],
        scratch_types=[
            pltpu.VMEM((4, CH), jnp.int32),
            pltpu.VMEM((4, CH), jnp.int32),
            pltpu.VMEM((CH, w_half), jnp.float32),
            pltpu.VMEM((CH, w_half), jnp.float32),
            pltpu.VMEM_SHARED((N + 8, w_half), jnp.float32),
            pltpu.SemaphoreType.DMA,
            pltpu.SemaphoreType.DMA,
            pltpu.SemaphoreType.DMA,
            pltpu.SemaphoreType.DMA,
            pltpu.SemaphoreType.DMA,
            pltpu.SemaphoreType.DMA,
            pltpu.SemaphoreType.DMA,
            pltpu.SemaphoreType.DMA,
        ],
    )
    def sc_kernel(hwa, hwb, src_r, dst_r, zblk, zblk8, ua, ub,
                  sidx, didx, rows0, rows1, ush,
                  si0, si1, si2, si3, sg0, sg1, ss0, ss1):
        c = lax.axis_index("c")
        s = lax.axis_index("s")
        pltpu.sync_copy(zblk, ush.at[pl.ds(s * NPT, NPT)])

        @pl.when(s == NTILES - 1)
        def _():
            pltpu.sync_copy(zblk8, ush.at[pl.ds(N, 8)])

        plsc.subcore_barrier()

        rows = (rows0, rows1)
        si = (si0, si1, si2, si3)
        sg = (sg0, sg1)
        ss = (ss0, ss1)

        def fetch_idx(k, u):
            pltpu.async_copy(src_r.at[s, k], sidx.at[u], si[u])
            pltpu.async_copy(dst_r.at[s, k], didx.at[u], si[u])

        def wait_idx(k, u):
            pltpu.make_async_copy(src_r.at[s, k], sidx.at[u], si[u]).wait()
            pltpu.make_async_copy(dst_r.at[s, k], didx.at[u], si[u]).wait()

        def run(hw_ref):
            def issue_gather(u, buf, sem):
                pltpu.async_copy(hw_ref.at[sidx.at[u, pl.ds(0, 64)]],
                                 buf.at[pl.ds(0, 64)], sem)
                pltpu.async_copy(hw_ref.at[sidx.at[u, pl.ds(64, 64)]],
                                 buf.at[pl.ds(64, 64)], sem)

            def wait_gather(u, buf, sem):
                pltpu.make_async_copy(hw_ref.at[sidx.at[u, pl.ds(0, 64)]],
                                      buf.at[pl.ds(0, 64)], sem).wait()
                pltpu.make_async_copy(hw_ref.at[sidx.at[u, pl.ds(64, 64)]],
                                      buf.at[pl.ds(64, 64)], sem).wait()

            fetch_idx(0, 0)
            fetch_idx(1, 1)
            wait_idx(0, 0)
            issue_gather(0, rows[0], sg[0])

            def body_fn(j, carry):
                for r in range(4):
                    k = 4 * j + r
                    b = r % 2
                    nb = 1 - b
                    wait_gather(r, rows[b], sg[b])

                    @pl.when(k >= 1)
                    def _():
                        pltpu.make_async_copy(
                            rows[nb], ush.at[didx.at[(r + 3) % 4]],
                            ss[nb]).wait()

                    @pl.when(k + 2 < K_PT)
                    def _():
                        fetch_idx(k + 2, (r + 2) % 4)

                    @pl.when(k + 1 < K_PT)
                    def _():
                        wait_idx(k + 1, (r + 1) % 4)
                        issue_gather((r + 1) % 4, rows[nb], sg[nb])

                    pltpu.async_copy(
                        rows[b], ush.at[didx.at[r]], ss[b], add=True)
                return carry

            lax.fori_loop(0, K_PT // 4, body_fn, 0)
            pltpu.make_async_copy(
                rows[1], ush.at[didx.at[3]], ss[1]).wait()

        @pl.when(c == 0)
        def _():
            run(hwa)

        @pl.when(c == 1)
        def _():
            run(hwb)

        plsc.subcore_barrier()

        @pl.when(c == 0)
        def _():
            pltpu.sync_copy(ush.at[pl.ds(s * NPT, NPT)],
                            ua.at[pl.ds(s * NPT, NPT)])

        @pl.when(c == 1)
        def _():
            pltpu.sync_copy(ush.at[pl.ds(s * NPT, NPT)],
                            ub.at[pl.ds(s * NPT, NPT)])

    return sc_kernel


def _combine_body(ua_ref, ub_ref, ed_ref, h_ref, w1t_ref, w2t_ref,
                  bs_ref, g_ref, be_ref, out_ref):
    d = h_ref.shape[1]
    u = jnp.concatenate([ua_ref[...], ub_ref[...]], axis=1)
    t = u[:, :d]
    s_sum = u[:, d:d + 1]
    ed = ed_ref[...]
    r = ed / jnp.maximum(ed * s_sum, 1e-8)
    agg = t * r
    z = (jnp.dot(h_ref[...], w1t_ref[...], preferred_element_type=jnp.float32)
         + jnp.dot(agg, w2t_ref[...], preferred_element_type=jnp.float32)
         + bs_ref[...])
    mu = jnp.mean(z, axis=1, keepdims=True)
    var = jnp.mean((z - mu) ** 2, axis=1, keepdims=True)
    out_ref[...] = jnp.maximum(
        (z - mu) * lax.rsqrt(var + 1e-5) * g_ref[...] + be_ref[...], 0.0)


def _make_combine(d, w_half):
    return pl.pallas_call(
        _combine_body,
        grid=(GRID,),
        in_specs=[
            pl.BlockSpec((BLK, w_half), lambda i: (i, 0)),
            pl.BlockSpec((BLK, w_half), lambda i: (i, 0)),
            pl.BlockSpec((BLK, 1), lambda i: (i, 0)),
            pl.BlockSpec((BLK, d), lambda i: (i, 0)),
            pl.BlockSpec((d, 256), lambda i: (0, 0)),
            pl.BlockSpec((d, 256), lambda i: (0, 0)),
            pl.BlockSpec((1, 256), lambda i: (0, 0)),
            pl.BlockSpec((1, 256), lambda i: (0, 0)),
            pl.BlockSpec((1, 256), lambda i: (0, 0)),
        ],
        out_specs=pl.BlockSpec((BLK, 256), lambda i: (i, 0)),
        out_shape=jax.ShapeDtypeStruct((N, 256), jnp.float32),
    )


def _combine_final_body(ua_ref, ub_ref, ed_ref, h_ref, w1t_ref, w2t_ref,
                        bs_ref, g_ref, be_ref, x_ref, wskipt_ref,
                        bskip_ref, acc_ref):
    d = h_ref.shape[1]
    u = jnp.concatenate([ua_ref[...], ub_ref[...]], axis=1)
    t = u[:, :d]
    s_sum = u[:, d:d + 1]
    ed = ed_ref[...]
    r = ed / jnp.maximum(ed * s_sum, 1e-8)
    agg = t * r
    z = (jnp.dot(h_ref[...], w1t_ref[...], preferred_element_type=jnp.float32)
         + jnp.dot(agg, w2t_ref[...], preferred_element_type=jnp.float32)
         + bs_ref[...])
    mu = jnp.mean(z, axis=1, keepdims=True)
    var = jnp.mean((z - mu) ** 2, axis=1, keepdims=True)
    h3 = jnp.maximum(
        (z - mu) * lax.rsqrt(var + 1e-5) * g_ref[...] + be_ref[...], 0.0)
    xs = jnp.sum(x_ref[...], axis=0, keepdims=True)
    part = (jnp.sum(h3, axis=0, keepdims=True)
            + jnp.dot(xs, wskipt_ref[...], preferred_element_type=jnp.float32))

    @pl.when(pl.program_id(0) == 0)
    def _():
        acc_ref[...] = jnp.zeros_like(acc_ref)

    acc_ref[...] += part

    @pl.when(pl.program_id(0) == GRID - 1)
    def _():
        acc_ref[...] = acc_ref[...] * (1.0 / N) + bskip_ref[...]


def _make_combine_final(d, w_half, in_dim):
    return pl.pallas_call(
        _combine_final_body,
        grid=(GRID,),
        in_specs=[
            pl.BlockSpec((BLK, w_half), lambda i: (i, 0)),
            pl.BlockSpec((BLK, w_half), lambda i: (i, 0)),
            pl.BlockSpec((BLK, 1), lambda i: (i, 0)),
            pl.BlockSpec((BLK, d), lambda i: (i, 0)),
            pl.BlockSpec((d, 256), lambda i: (0, 0)),
            pl.BlockSpec((d, 256), lambda i: (0, 0)),
            pl.BlockSpec((1, 256), lambda i: (0, 0)),
            pl.BlockSpec((1, 256), lambda i: (0, 0)),
            pl.BlockSpec((1, 256), lambda i: (0, 0)),
            pl.BlockSpec((BLK, in_dim), lambda i: (i, 0)),
            pl.BlockSpec((in_dim, 256), lambda i: (0, 0)),
            pl.BlockSpec((1, 256), lambda i: (0, 0)),
        ],
        out_specs=pl.BlockSpec((1, 256), lambda i: (0, 0)),
        out_shape=jax.ShapeDtypeStruct((1, 256), jnp.float32),
    )


def kernel(x, edge_index, Ws0, bs0, g0, be0, Wa0, ba0, Ws1, bs1, g1, be1,
           Wa1, ba1, Ws2, bs2, g2, be2, Wa2, ba2, Wskip, bskip):
    pad = E_PAD - E
    src3 = jnp.concatenate(
        [edge_index[0], jnp.zeros((pad,), jnp.int32)]).reshape(
            NTILES, K_PT, CH)
    dst3 = jnp.concatenate(
        [edge_index[1], jnp.full((pad,), N, jnp.int32)]).reshape(
            NTILES, K_PT, CH)

    dims = (128, 256, 256)
    halves = (80, 144, 144)
    params = ((Ws0, bs0, g0, be0, Wa0, ba0),
              (Ws1, bs1, g1, be1, Wa1, ba1),
              (Ws2, bs2, g2, be2, Wa2, ba2))

    h = x
    for i in range(3):
        d = dims[i]
        w_half = halves[i]
        Ws, bs, g, be, Wa, ba = params[i]
        wa2 = jnp.stack([Wa[0, :d], Wa[0, d:]], axis=1)
        ba_arr = ba.reshape(1, 1)
        hwa, hwb, ed = _make_prep(d, w_half)(h, wa2, ba_arr)
        zblk = jnp.zeros((NPT, w_half), jnp.float32)
        zblk8 = jnp.zeros((8, w_half), jnp.float32)
        ua, ub = _make_sc(w_half)(hwa, hwb, src3, dst3, zblk, zblk8)
        w1t = Ws[:, :d].T
        w2t = Ws[:, d:].T
        if i < 2:
            h = _make_combine(d, w_half)(
                ua, ub, ed, h, w1t, w2t,
                bs.reshape(1, 256), g.reshape(1, 256), be.reshape(1, 256))
        else:
            acc = _make_combine_final(d, w_half, 128)(
                ua, ub, ed, h, w1t, w2t,
                bs.reshape(1, 256), g.reshape(1, 256), be.reshape(1, 256),
                x, Wskip.T, bskip.reshape(1, 256))
    return acc.reshape(256)


# speedup vs baseline: 1.8293x; 1.6815x over previous
import functools

import jax
import jax.numpy as jnp
from jax import lax
from jax.experimental import pallas as pl
from jax.experimental.pallas import tpu as pltpu
from jax.experimental.pallas import tpu_sc as plsc

N = 10000
E = 320000
CH = 128
NTILES = 16
NPT = N // NTILES
K_PT = 160
E_PAD = NTILES * K_PT * CH
BLK = 1000
GRID = N // BLK


def _prep_body(h_ref, wa_ref, ba_ref, *out_refs):
    ed_ref = out_refs[-1]
    tbl_refs = out_refs[:-1]
    h = h_ref[...]
    d = h.shape[1]
    w_grp = tbl_refs[0].shape[1]
    a = jnp.dot(h, wa_ref[...], preferred_element_type=jnp.float32)
    a_dst = a[:, 0:1]
    a_src = a[:, 1:2]
    w = jnp.exp(a_src)
    ed_ref[...] = jnp.exp(a_dst + ba_ref[0, 0])
    hw = h * w
    g = len(tbl_refs)
    pad = jnp.concatenate(
        [hw, w, jnp.zeros((h.shape[0], g * w_grp - d - 1), jnp.float32)],
        axis=1)
    for i, ref in enumerate(tbl_refs):
        ref[...] = pad[:, i * w_grp:(i + 1) * w_grp]


def _make_prep(d, w_grp, g):
    return pl.pallas_call(
        _prep_body,
        grid=(GRID,),
        in_specs=[
            pl.BlockSpec((BLK, d), lambda i: (i, 0)),
            pl.BlockSpec((d, 2), lambda i: (0, 0)),
            pl.BlockSpec((1, 1), lambda i: (0, 0)),
        ],
        out_specs=[pl.BlockSpec((BLK, w_grp), lambda i: (i, 0))] * g
        + [pl.BlockSpec((BLK, 1), lambda i: (i, 0))],
        out_shape=[jax.ShapeDtypeStruct((N, w_grp), jnp.float32)] * g
        + [jax.ShapeDtypeStruct((N, 1), jnp.float32)],
    )


def _make_sc(w_grp, g):
    mesh = plsc.VectorSubcoreMesh(core_axis_name="c", subcore_axis_name="s")
    g2 = g // 2

    @functools.partial(
        pl.kernel,
        mesh=mesh,
        compiler_params=pltpu.CompilerParams(use_tc_tiling_on_sc=False),
        out_type=[jax.ShapeDtypeStruct((N, w_grp), jnp.float32)] * g,
        scratch_types=[
            pltpu.VMEM((4, CH), jnp.int32),
            pltpu.VMEM((4, CH), jnp.int32),
            pltpu.VMEM((CH, w_grp), jnp.float32),
            pltpu.VMEM((CH, w_grp), jnp.float32),
            pltpu.VMEM_SHARED((N, w_grp), jnp.float32),
            pltpu.VMEM_SHARED((N + 8, w_grp), jnp.float32),
            pltpu.SemaphoreType.DMA,
            pltpu.SemaphoreType.DMA,
            pltpu.SemaphoreType.DMA,
            pltpu.SemaphoreType.DMA,
            pltpu.SemaphoreType.DMA,
            pltpu.SemaphoreType.DMA,
            pltpu.SemaphoreType.DMA,
            pltpu.SemaphoreType.DMA,
        ],
    )
    def sc_kernel(*refs):
        ins = refs[:g + 4]
        tbls = ins[:g]
        src_r, dst_r, zblk, zblk8 = ins[g:]
        outs = refs[g + 4:2 * g + 4]
        (sidx, didx, rows0, rows1, tbl, ush,
         si0, si1, si2, si3, sg0, sg1, ss0, ss1) = refs[2 * g + 4:]

        c = lax.axis_index("c")
        s = lax.axis_index("s")
        slab = pl.ds(s * NPT, NPT)

        rows = (rows0, rows1)
        si = (si0, si1, si2, si3)
        sg = (sg0, sg1)
        ss = (ss0, ss1)

        def fetch_idx(k, u):
            pltpu.async_copy(src_r.at[s, k], sidx.at[u], si[u])
            pltpu.async_copy(dst_r.at[s, k], didx.at[u], si[u])

        def wait_idx(k, u):
            pltpu.make_async_copy(src_r.at[s, k], sidx.at[u], si[u]).wait()
            pltpu.make_async_copy(dst_r.at[s, k], didx.at[u], si[u]).wait()

        def run_pass(tbl_hbm, out_hbm):
            pltpu.sync_copy(tbl_hbm.at[slab], tbl.at[slab])
            pltpu.sync_copy(zblk, ush.at[slab])

            @pl.when(s == NTILES - 1)
            def _():
                pltpu.sync_copy(zblk8, ush.at[pl.ds(N, 8)])

            plsc.subcore_barrier()

            fetch_idx(0, 0)
            fetch_idx(1, 1)
            wait_idx(0, 0)
            pltpu.async_copy(tbl.at[sidx.at[0]], rows[0], sg[0])

            def body_fn(j, carry):
                for r in range(4):
                    k = 4 * j + r
                    b = r % 2
                    nb = 1 - b
                    pltpu.make_async_copy(
                        tbl.at[sidx.at[r]], rows[b], sg[b]).wait()

                    @pl.when(k >= 1)
                    def _():
                        pltpu.make_async_copy(
                            rows[nb], ush.at[didx.at[(r + 3) % 4]],
                            ss[nb]).wait()

                    @pl.when(k + 2 < K_PT)
                    def _():
                        fetch_idx(k + 2, (r + 2) % 4)

                    @pl.when(k + 1 < K_PT)
                    def _():
                        wait_idx(k + 1, (r + 1) % 4)
                        pltpu.async_copy(
                            tbl.at[sidx.at[(r + 1) % 4]], rows[nb], sg[nb])

                    pltpu.async_copy(
                        rows[b], ush.at[didx.at[r]], ss[b], add=True)
                return carry

            lax.fori_loop(0, K_PT // 4, body_fn, 0)
            pltpu.make_async_copy(
                rows[1], ush.at[didx.at[3]], ss[1]).wait()

            plsc.subcore_barrier()
            pltpu.sync_copy(ush.at[slab], out_hbm.at[slab])

        @pl.when(c == 0)
        def _():
            for i in range(g2):
                run_pass(tbls[i], outs[i])

        @pl.when(c == 1)
        def _():
            for i in range(g2):
                run_pass(tbls[g2 + i], outs[g2 + i])

    return sc_kernel


def _ln_relu(z, g_ref, be_ref):
    mu = jnp.mean(z, axis=1, keepdims=True)
    var = jnp.mean((z - mu) ** 2, axis=1, keepdims=True)
    return jnp.maximum(
        (z - mu) * lax.rsqrt(var + 1e-5) * g_ref[...] + be_ref[...], 0.0)


def _combine_body(g, d, *refs):
    u_refs = refs[:g]
    (ed_ref, h_ref, w1t_ref, w2t_ref, bs_ref, g_ref, be_ref, out_ref) = \
        refs[g:]
    u = jnp.concatenate([r[...] for r in u_refs], axis=1)
    t = u[:, :d]
    s_sum = u[:, d:d + 1]
    ed = ed_ref[...]
    r = ed / jnp.maximum(ed * s_sum, 1e-8)
    agg = t * r
    z = (jnp.dot(h_ref[...], w1t_ref[...], preferred_element_type=jnp.float32)
         + jnp.dot(agg, w2t_ref[...], preferred_element_type=jnp.float32)
         + bs_ref[...])
    out_ref[...] = _ln_relu(z, g_ref, be_ref)


def _make_combine(d, w_grp, g):
    return pl.pallas_call(
        functools.partial(_combine_body, g, d),
        grid=(GRID,),
        in_specs=[pl.BlockSpec((BLK, w_grp), lambda i: (i, 0))] * g + [
            pl.BlockSpec((BLK, 1), lambda i: (i, 0)),
            pl.BlockSpec((BLK, d), lambda i: (i, 0)),
            pl.BlockSpec((d, 256), lambda i: (0, 0)),
            pl.BlockSpec((d, 256), lambda i: (0, 0)),
            pl.BlockSpec((1, 256), lambda i: (0, 0)),
            pl.BlockSpec((1, 256), lambda i: (0, 0)),
            pl.BlockSpec((1, 256), lambda i: (0, 0)),
        ],
        out_specs=pl.BlockSpec((BLK, 256), lambda i: (i, 0)),
        out_shape=jax.ShapeDtypeStruct((N, 256), jnp.float32),
    )


def _combine_final_body(g, d, *refs):
    u_refs = refs[:g]
    (ed_ref, h_ref, w1t_ref, w2t_ref, bs_ref, g_ref, be_ref, x_ref,
     wskipt_ref, bskip_ref, acc_ref) = refs[g:]
    u = jnp.concatenate([r[...] for r in u_refs], axis=1)
    t = u[:, :d]
    s_sum = u[:, d:d + 1]
    ed = ed_ref[...]
    r = ed / jnp.maximum(ed * s_sum, 1e-8)
    agg = t * r
    z = (jnp.dot(h_ref[...], w1t_ref[...], preferred_element_type=jnp.float32)
         + jnp.dot(agg, w2t_ref[...], preferred_element_type=jnp.float32)
         + bs_ref[...])
    h3 = _ln_relu(z, g_ref, be_ref)
    xs = jnp.sum(x_ref[...], axis=0, keepdims=True)
    part = (jnp.sum(h3, axis=0, keepdims=True)
            + jnp.dot(xs, wskipt_ref[...], preferred_element_type=jnp.float32))

    @pl.when(pl.program_id(0) == 0)
    def _():
        acc_ref[...] = jnp.zeros_like(acc_ref)

    acc_ref[...] += part

    @pl.when(pl.program_id(0) == GRID - 1)
    def _():
        acc_ref[...] = acc_ref[...] * (1.0 / N) + bskip_ref[...]


def _make_combine_final(d, w_grp, g, in_dim):
    return pl.pallas_call(
        functools.partial(_combine_final_body, g, d),
        grid=(GRID,),
        in_specs=[pl.BlockSpec((BLK, w_grp), lambda i: (i, 0))] * g + [
            pl.BlockSpec((BLK, 1), lambda i: (i, 0)),
            pl.BlockSpec((BLK, d), lambda i: (i, 0)),
            pl.BlockSpec((d, 256), lambda i: (0, 0)),
            pl.BlockSpec((d, 256), lambda i: (0, 0)),
            pl.BlockSpec((1, 256), lambda i: (0, 0)),
            pl.BlockSpec((1, 256), lambda i: (0, 0)),
            pl.BlockSpec((1, 256), lambda i: (0, 0)),
            pl.BlockSpec((BLK, in_dim), lambda i: (i, 0)),
            pl.BlockSpec((in_dim, 256), lambda i: (0, 0)),
            pl.BlockSpec((1, 256), lambda i: (0, 0)),
        ],
        out_specs=pl.BlockSpec((1, 256), lambda i: (0, 0)),
        out_shape=jax.ShapeDtypeStruct((1, 256), jnp.float32),
    )


def kernel(x, edge_index, Ws0, bs0, g0, be0, Wa0, ba0, Ws1, bs1, g1, be1,
           Wa1, ba1, Ws2, bs2, g2, be2, Wa2, ba2, Wskip, bskip):
    pad = E_PAD - E
    src3 = jnp.concatenate(
        [edge_index[0], jnp.zeros((pad,), jnp.int32)]).reshape(
            NTILES, K_PT, CH)
    dst3 = jnp.concatenate(
        [edge_index[1], jnp.full((pad,), N, jnp.int32)]).reshape(
            NTILES, K_PT, CH)

    dims = (128, 256, 256)
    grouping = ((80, 2), (72, 4), (72, 4))
    params = ((Ws0, bs0, g0, be0, Wa0, ba0),
              (Ws1, bs1, g1, be1, Wa1, ba1),
              (Ws2, bs2, g2, be2, Wa2, ba2))

    h = x
    for i in range(3):
        d = dims[i]
        w_grp, g = grouping[i]
        Ws, bs, gg, be, Wa, ba = params[i]
        wa2 = jnp.stack([Wa[0, :d], Wa[0, d:]], axis=1)
        ba_arr = ba.reshape(1, 1)
        outs = _make_prep(d, w_grp, g)(h, wa2, ba_arr)
        tbls, ed = outs[:-1], outs[-1]
        zblk = jnp.zeros((NPT, w_grp), jnp.float32)
        zblk8 = jnp.zeros((8, w_grp), jnp.float32)
        us = _make_sc(w_grp, g)(*tbls, src3, dst3, zblk, zblk8)
        w1t = Ws[:, :d].T
        w2t = Ws[:, d:].T
        if i < 2:
            h = _make_combine(d, w_grp, g)(
                *us, ed, h, w1t, w2t,
                bs.reshape(1, 256), gg.reshape(1, 256), be.reshape(1, 256))
        else:
            acc = _make_combine_final(d, w_grp, g, 128)(
                *us, ed, h, w1t, w2t,
                bs.reshape(1, 256), gg.reshape(1, 256), be.reshape(1, 256),
                x, Wskip.T, bskip.reshape(1, 256))
    return acc.reshape(256)


# speedup vs baseline: 1.8723x; 1.0235x over previous
import functools

import jax
import jax.numpy as jnp
from jax import lax
from jax.experimental import pallas as pl
from jax.experimental.pallas import tpu as pltpu
from jax.experimental.pallas import tpu_sc as plsc

N = 10000
E = 320000
CH = 128
NS = 2
NTILES = 16
NPT = N // NTILES
K_PT = 80
E_PAD = NTILES * K_PT * NS * CH
BLK = 1000
GRID = N // BLK


def _prep_body(h_ref, wa_ref, ba_ref, *out_refs):
    ed_ref = out_refs[-1]
    tbl_refs = out_refs[:-1]
    h = h_ref[...]
    d = h.shape[1]
    w_grp = tbl_refs[0].shape[1]
    a = jnp.dot(h, wa_ref[...], preferred_element_type=jnp.float32)
    a_dst = a[:, 0:1]
    a_src = a[:, 1:2]
    w = jnp.exp(a_src)
    ed_ref[...] = jnp.exp(a_dst + ba_ref[0, 0])
    hw = h * w
    g = len(tbl_refs)
    pad = jnp.concatenate(
        [hw, w, jnp.zeros((h.shape[0], g * w_grp - d - 1), jnp.float32)],
        axis=1)
    for i, ref in enumerate(tbl_refs):
        ref[...] = pad[:, i * w_grp:(i + 1) * w_grp]


def _make_prep(d, w_grp, g):
    return pl.pallas_call(
        _prep_body,
        grid=(GRID,),
        in_specs=[
            pl.BlockSpec((BLK, d), lambda i: (i, 0)),
            pl.BlockSpec((d, 2), lambda i: (0, 0)),
            pl.BlockSpec((1, 1), lambda i: (0, 0)),
        ],
        out_specs=[pl.BlockSpec((BLK, w_grp), lambda i: (i, 0))] * g
        + [pl.BlockSpec((BLK, 1), lambda i: (i, 0))],
        out_shape=[jax.ShapeDtypeStruct((N, w_grp), jnp.float32)] * g
        + [jax.ShapeDtypeStruct((N, 1), jnp.float32)],
    )


def _make_sc(w_grp, g):
    mesh = plsc.VectorSubcoreMesh(core_axis_name="c", subcore_axis_name="s")
    g2 = g // 2

    @functools.partial(
        pl.kernel,
        mesh=mesh,
        compiler_params=pltpu.CompilerParams(use_tc_tiling_on_sc=False),
        out_type=[jax.ShapeDtypeStruct((N, w_grp), jnp.float32)] * g,
        scratch_types=[
            pltpu.VMEM((4, NS, CH), jnp.int32),
            pltpu.VMEM((4, NS, CH), jnp.int32),
            pltpu.VMEM((NS * CH, w_grp), jnp.float32),
            pltpu.VMEM((NS * CH, w_grp), jnp.float32),
            pltpu.VMEM_SHARED((N, w_grp), jnp.float32),
            pltpu.VMEM_SHARED((N + 8, w_grp), jnp.float32),
            pltpu.SemaphoreType.DMA,
            pltpu.SemaphoreType.DMA,
            pltpu.SemaphoreType.DMA,
            pltpu.SemaphoreType.DMA,
            pltpu.SemaphoreType.DMA,
            pltpu.SemaphoreType.DMA,
            pltpu.SemaphoreType.DMA,
            pltpu.SemaphoreType.DMA,
        ],
    )
    def sc_kernel(*refs):
        ins = refs[:g + 4]
        tbls = ins[:g]
        src_r, dst_r, zblk, zblk8 = ins[g:]
        outs = refs[g + 4:2 * g + 4]
        (sidx, didx, rows0, rows1, tbl, ush,
         si0, si1, si2, si3, sg0, sg1, ss0, ss1) = refs[2 * g + 4:]

        c = lax.axis_index("c")
        s = lax.axis_index("s")
        slab = pl.ds(s * NPT, NPT)

        rows = (rows0, rows1)
        si = (si0, si1, si2, si3)
        sg = (sg0, sg1)
        ss = (ss0, ss1)

        def fetch_idx(k, u):
            pltpu.async_copy(src_r.at[s, k], sidx.at[u], si[u])
            pltpu.async_copy(dst_r.at[s, k], didx.at[u], si[u])

        def wait_idx(k, u):
            pltpu.make_async_copy(src_r.at[s, k], sidx.at[u], si[u]).wait()
            pltpu.make_async_copy(dst_r.at[s, k], didx.at[u], si[u]).wait()

        def issue_gather(tbl_ref, u, buf, sem):
            for q in range(NS):
                pltpu.async_copy(tbl_ref.at[sidx.at[u, q]],
                                 buf.at[pl.ds(q * CH, CH)], sem)

        def wait_gather(tbl_ref, u, buf, sem):
            for q in range(NS):
                pltpu.make_async_copy(tbl_ref.at[sidx.at[u, q]],
                                      buf.at[pl.ds(q * CH, CH)], sem).wait()

        def issue_scatter(u, buf, sem):
            for q in range(NS):
                pltpu.async_copy(buf.at[pl.ds(q * CH, CH)],
                                 ush.at[didx.at[u, q]], sem, add=True)

        def wait_scatter(u, buf, sem):
            for q in range(NS):
                pltpu.make_async_copy(buf.at[pl.ds(q * CH, CH)],
                                      ush.at[didx.at[u, q]], sem).wait()

        def run_pass(tbl_hbm, out_hbm):
            pltpu.sync_copy(tbl_hbm.at[slab], tbl.at[slab])
            pltpu.sync_copy(zblk, ush.at[slab])

            @pl.when(s == NTILES - 1)
            def _():
                pltpu.sync_copy(zblk8, ush.at[pl.ds(N, 8)])

            plsc.subcore_barrier()

            fetch_idx(0, 0)
            fetch_idx(1, 1)
            wait_idx(0, 0)
            issue_gather(tbl, 0, rows[0], sg[0])

            def body_fn(j, carry):
                for r in range(4):
                    k = 4 * j + r
                    b = r % 2
                    nb = 1 - b
                    wait_gather(tbl, r, rows[b], sg[b])

                    @pl.when(k >= 1)
                    def _():
                        wait_scatter((r + 3) % 4, rows[nb], ss[nb])

                    @pl.when(k + 2 < K_PT)
                    def _():
                        fetch_idx(k + 2, (r + 2) % 4)

                    @pl.when(k + 1 < K_PT)
                    def _():
                        wait_idx(k + 1, (r + 1) % 4)
                        issue_gather(tbl, (r + 1) % 4, rows[nb], sg[nb])

                    issue_scatter(r, rows[b], ss[b])
                return carry

            lax.fori_loop(0, K_PT // 4, body_fn, 0)
            wait_scatter(3, rows[1], ss[1])

            plsc.subcore_barrier()
            pltpu.sync_copy(ush.at[slab], out_hbm.at[slab])

        @pl.when(c == 0)
        def _():
            for i in range(g2):
                run_pass(tbls[i], outs[i])

        @pl.when(c == 1)
        def _():
            for i in range(g2):
                run_pass(tbls[g2 + i], outs[g2 + i])

    return sc_kernel


def _ln_relu(z, g_ref, be_ref):
    mu = jnp.mean(z, axis=1, keepdims=True)
    var = jnp.mean((z - mu) ** 2, axis=1, keepdims=True)
    return jnp.maximum(
        (z - mu) * lax.rsqrt(var + 1e-5) * g_ref[...] + be_ref[...], 0.0)


def _combine_body(g, d, *refs):
    u_refs = refs[:g]
    (ed_ref, h_ref, w1t_ref, w2t_ref, bs_ref, g_ref, be_ref, out_ref) = \
        refs[g:]
    u = jnp.concatenate([r[...] for r in u_refs], axis=1)
    t = u[:, :d]
    s_sum = u[:, d:d + 1]
    ed = ed_ref[...]
    r = ed / jnp.maximum(ed * s_sum, 1e-8)
    agg = t * r
    z = (jnp.dot(h_ref[...], w1t_ref[...], preferred_element_type=jnp.float32)
         + jnp.dot(agg, w2t_ref[...], preferred_element_type=jnp.float32)
         + bs_ref[...])
    out_ref[...] = _ln_relu(z, g_ref, be_ref)


def _make_combine(d, w_grp, g):
    return pl.pallas_call(
        functools.partial(_combine_body, g, d),
        grid=(GRID,),
        in_specs=[pl.BlockSpec((BLK, w_grp), lambda i: (i, 0))] * g + [
            pl.BlockSpec((BLK, 1), lambda i: (i, 0)),
            pl.BlockSpec((BLK, d), lambda i: (i, 0)),
            pl.BlockSpec((d, 256), lambda i: (0, 0)),
            pl.BlockSpec((d, 256), lambda i: (0, 0)),
            pl.BlockSpec((1, 256), lambda i: (0, 0)),
            pl.BlockSpec((1, 256), lambda i: (0, 0)),
            pl.BlockSpec((1, 256), lambda i: (0, 0)),
        ],
        out_specs=pl.BlockSpec((BLK, 256), lambda i: (i, 0)),
        out_shape=jax.ShapeDtypeStruct((N, 256), jnp.float32),
    )


def _combine_final_body(g, d, *refs):
    u_refs = refs[:g]
    (ed_ref, h_ref, w1t_ref, w2t_ref, bs_ref, g_ref, be_ref, x_ref,
     wskipt_ref, bskip_ref, acc_ref) = refs[g:]
    u = jnp.concatenate([r[...] for r in u_refs], axis=1)
    t = u[:, :d]
    s_sum = u[:, d:d + 1]
    ed = ed_ref[...]
    r = ed / jnp.maximum(ed * s_sum, 1e-8)
    agg = t * r
    z = (jnp.dot(h_ref[...], w1t_ref[...], preferred_element_type=jnp.float32)
         + jnp.dot(agg, w2t_ref[...], preferred_element_type=jnp.float32)
         + bs_ref[...])
    h3 = _ln_relu(z, g_ref, be_ref)
    xs = jnp.sum(x_ref[...], axis=0, keepdims=True)
    part = (jnp.sum(h3, axis=0, keepdims=True)
            + jnp.dot(xs, wskipt_ref[...], preferred_element_type=jnp.float32))

    @pl.when(pl.program_id(0) == 0)
    def _():
        acc_ref[...] = jnp.zeros_like(acc_ref)

    acc_ref[...] += part

    @pl.when(pl.program_id(0) == GRID - 1)
    def _():
        acc_ref[...] = acc_ref[...] * (1.0 / N) + bskip_ref[...]


def _make_combine_final(d, w_grp, g, in_dim):
    return pl.pallas_call(
        functools.partial(_combine_final_body, g, d),
        grid=(GRID,),
        in_specs=[pl.BlockSpec((BLK, w_grp), lambda i: (i, 0))] * g + [
            pl.BlockSpec((BLK, 1), lambda i: (i, 0)),
            pl.BlockSpec((BLK, d), lambda i: (i, 0)),
            pl.BlockSpec((d, 256), lambda i: (0, 0)),
            pl.BlockSpec((d, 256), lambda i: (0, 0)),
            pl.BlockSpec((1, 256), lambda i: (0, 0)),
            pl.BlockSpec((1, 256), lambda i: (0, 0)),
            pl.BlockSpec((1, 256), lambda i: (0, 0)),
            pl.BlockSpec((BLK, in_dim), lambda i: (i, 0)),
            pl.BlockSpec((in_dim, 256), lambda i: (0, 0)),
            pl.BlockSpec((1, 256), lambda i: (0, 0)),
        ],
        out_specs=pl.BlockSpec((1, 256), lambda i: (0, 0)),
        out_shape=jax.ShapeDtypeStruct((1, 256), jnp.float32),
    )


def kernel(x, edge_index, Ws0, bs0, g0, be0, Wa0, ba0, Ws1, bs1, g1, be1,
           Wa1, ba1, Ws2, bs2, g2, be2, Wa2, ba2, Wskip, bskip):
    pad = E_PAD - E
    src3 = jnp.concatenate(
        [edge_index[0], jnp.zeros((pad,), jnp.int32)]).reshape(
            NTILES, K_PT, NS, CH)
    dst3 = jnp.concatenate(
        [edge_index[1], jnp.full((pad,), N, jnp.int32)]).reshape(
            NTILES, K_PT, NS, CH)

    dims = (128, 256, 256)
    grouping = ((72, 2), (72, 4), (72, 4))
    params = ((Ws0, bs0, g0, be0, Wa0, ba0),
              (Ws1, bs1, g1, be1, Wa1, ba1),
              (Ws2, bs2, g2, be2, Wa2, ba2))

    h = x
    for i in range(3):
        d = dims[i]
        w_grp, g = grouping[i]
        Ws, bs, gg, be, Wa, ba = params[i]
        wa2 = jnp.stack([Wa[0, :d], Wa[0, d:]], axis=1)
        ba_arr = ba.reshape(1, 1)
        outs = _make_prep(d, w_grp, g)(h, wa2, ba_arr)
        tbls, ed = outs[:-1], outs[-1]
        zblk = jnp.zeros((NPT, w_grp), jnp.float32)
        zblk8 = jnp.zeros((8, w_grp), jnp.float32)
        us = _make_sc(w_grp, g)(*tbls, src3, dst3, zblk, zblk8)
        w1t = Ws[:, :d].T
        w2t = Ws[:, d:].T
        if i < 2:
            h = _make_combine(d, w_grp, g)(
                *us, ed, h, w1t, w2t,
                bs.reshape(1, 256), gg.reshape(1, 256), be.reshape(1, 256))
        else:
            acc = _make_combine_final(d, w_grp, g, 128)(
                *us, ed, h, w1t, w2t,
                bs.reshape(1, 256), gg.reshape(1, 256), be.reshape(1, 256),
                x, Wskip.T, bskip.reshape(1, 256))
    return acc.reshape(256)


# speedup vs baseline: 1.9012x; 1.0154x over previous
import functools

import jax
import jax.numpy as jnp
from jax import lax
from jax.experimental import pallas as pl
from jax.experimental.pallas import tpu as pltpu
from jax.experimental.pallas import tpu_sc as plsc

N = 10000
E = 320000
CH = 128
NS = 2
NTILES = 16
NPT = N // NTILES
K_PT = 80
E_PAD = NTILES * K_PT * NS * CH
BLK = 1000
GRID = N // BLK


def _prep_body(h_ref, wa_ref, ba_ref, *out_refs):
    ed_ref = out_refs[-1]
    tbl_refs = out_refs[:-1]
    h = h_ref[...]
    d = h.shape[1]
    w_grp = tbl_refs[0].shape[1]
    a = jnp.dot(h, wa_ref[...], preferred_element_type=jnp.float32)
    a_dst = a[:, 0:1]
    a_src = a[:, 1:2]
    w = jnp.exp(a_src)
    ed_ref[...] = jnp.exp(a_dst + ba_ref[0, 0])
    hw = h * w
    g = len(tbl_refs)
    pad = jnp.concatenate(
        [hw, w, jnp.zeros((h.shape[0], g * w_grp - d - 1), jnp.float32)],
        axis=1)
    for i, ref in enumerate(tbl_refs):
        ref[...] = pad[:, i * w_grp:(i + 1) * w_grp]


def _make_prep(d, w_grp, g):
    return pl.pallas_call(
        _prep_body,
        grid=(GRID,),
        in_specs=[
            pl.BlockSpec((BLK, d), lambda i: (i, 0)),
            pl.BlockSpec((d, 2), lambda i: (0, 0)),
            pl.BlockSpec((1, 1), lambda i: (0, 0)),
        ],
        out_specs=[pl.BlockSpec((BLK, w_grp), lambda i: (i, 0))] * g
        + [pl.BlockSpec((BLK, 1), lambda i: (i, 0))],
        out_shape=[jax.ShapeDtypeStruct((N, w_grp), jnp.float32)] * g
        + [jax.ShapeDtypeStruct((N, 1), jnp.float32)],
    )


def _make_sc(w_grp, g):
    mesh = plsc.VectorSubcoreMesh(core_axis_name="c", subcore_axis_name="s")
    g2 = g // 2

    @functools.partial(
        pl.kernel,
        mesh=mesh,
        compiler_params=pltpu.CompilerParams(use_tc_tiling_on_sc=False),
        out_type=[jax.ShapeDtypeStruct((N, w_grp), jnp.float32)] * g,
        scratch_types=[
            pltpu.VMEM((4, NS, CH), jnp.int32),
            pltpu.VMEM((4, NS, CH), jnp.int32),
            pltpu.VMEM((NS * CH, w_grp), jnp.float32),
            pltpu.VMEM((NS * CH, w_grp), jnp.float32),
            pltpu.VMEM_SHARED((N, w_grp), jnp.float32),
            pltpu.VMEM_SHARED((N + 8, w_grp), jnp.float32),
            pltpu.SemaphoreType.DMA,
            pltpu.SemaphoreType.DMA,
            pltpu.SemaphoreType.DMA,
            pltpu.SemaphoreType.DMA,
            pltpu.SemaphoreType.DMA,
            pltpu.SemaphoreType.DMA,
            pltpu.SemaphoreType.DMA,
            pltpu.SemaphoreType.DMA,
        ],
    )
    def sc_kernel(*refs):
        ins = refs[:g + 4]
        tbls = ins[:g]
        src_r, dst_r, zblk, zblk8 = ins[g:]
        outs = refs[g + 4:2 * g + 4]
        (sidx, didx, rows0, rows1, tbl, ush,
         si0, si1, si2, si3, sg0, sg1, ss0, ss1) = refs[2 * g + 4:]

        c = lax.axis_index("c")
        s = lax.axis_index("s")
        slab = pl.ds(s * NPT, NPT)

        rows = (rows0, rows1)
        si = (si0, si1, si2, si3)
        sg = (sg0, sg1)
        ss = (ss0, ss1)

        def fetch_idx(k, u):
            pltpu.async_copy(src_r.at[s, k], sidx.at[u], si[u])
            pltpu.async_copy(dst_r.at[s, k], didx.at[u], si[u])

        def wait_idx(k, u):
            pltpu.make_async_copy(src_r.at[s, k], sidx.at[u], si[u]).wait()
            pltpu.make_async_copy(dst_r.at[s, k], didx.at[u], si[u]).wait()

        def issue_gather(tbl_ref, u, buf, sem):
            for q in range(NS):
                pltpu.async_copy(tbl_ref.at[sidx.at[u, q]],
                                 buf.at[pl.ds(q * CH, CH)], sem)

        def wait_gather(tbl_ref, u, buf, sem):
            for q in range(NS):
                pltpu.make_async_copy(tbl_ref.at[sidx.at[u, q]],
                                      buf.at[pl.ds(q * CH, CH)], sem).wait()

        def issue_scatter(u, buf, sem):
            for q in range(NS):
                pltpu.async_copy(buf.at[pl.ds(q * CH, CH)],
                                 ush.at[didx.at[u, q]], sem, add=True)

        def wait_scatter(u, buf, sem):
            for q in range(NS):
                pltpu.make_async_copy(buf.at[pl.ds(q * CH, CH)],
                                      ush.at[didx.at[u, q]], sem).wait()

        def run_pass(tbl_hbm, out_hbm):
            pltpu.sync_copy(tbl_hbm.at[slab], tbl.at[slab])
            pltpu.sync_copy(zblk, ush.at[slab])

            @pl.when(s == NTILES - 1)
            def _():
                pltpu.sync_copy(zblk8, ush.at[pl.ds(N, 8)])

            plsc.subcore_barrier()

            fetch_idx(0, 0)
            fetch_idx(1, 1)
            wait_idx(0, 0)
            issue_gather(tbl, 0, rows[0], sg[0])

            def body_fn(j, carry):
                for r in range(4):
                    k = 4 * j + r
                    b = r % 2
                    nb = 1 - b
                    wait_gather(tbl, r, rows[b], sg[b])

                    @pl.when(k >= 1)
                    def _():
                        wait_scatter((r + 3) % 4, rows[nb], ss[nb])

                    @pl.when(k + 2 < K_PT)
                    def _():
                        fetch_idx(k + 2, (r + 2) % 4)

                    @pl.when(k + 1 < K_PT)
                    def _():
                        wait_idx(k + 1, (r + 1) % 4)
                        issue_gather(tbl, (r + 1) % 4, rows[nb], sg[nb])

                    issue_scatter(r, rows[b], ss[b])
                return carry

            lax.fori_loop(0, K_PT // 4, body_fn, 0)
            wait_scatter(3, rows[1], ss[1])

            plsc.subcore_barrier()
            pltpu.sync_copy(ush.at[slab], out_hbm.at[slab])

        @pl.when(c == 0)
        def _():
            for i in range(g2):
                run_pass(tbls[i], outs[i])

        @pl.when(c == 1)
        def _():
            for i in range(g2):
                run_pass(tbls[g2 + i], outs[g2 + i])

    return sc_kernel


def _ln_relu(z, g_ref, be_ref):
    mu = jnp.mean(z, axis=1, keepdims=True)
    var = jnp.mean((z - mu) ** 2, axis=1, keepdims=True)
    return jnp.maximum(
        (z - mu) * lax.rsqrt(var + 1e-5) * g_ref[...] + be_ref[...], 0.0)


def _combine_prep_body(g, d, g_next, w_next, *refs):
    u_refs = refs[:g]
    (ed_ref, h_ref, w1t_ref, w2t_ref, bs_ref, g_ref, be_ref,
     wa2n_ref, ban_ref) = refs[g:g + 9]
    h_out = refs[g + 9]
    tbl_outs = refs[g + 10:g + 10 + g_next]
    edn_ref = refs[g + 10 + g_next]
    u = jnp.concatenate([r[...] for r in u_refs], axis=1)
    t = u[:, :d]
    s_sum = u[:, d:d + 1]
    ed = ed_ref[...]
    r = ed / jnp.maximum(ed * s_sum, 1e-8)
    agg = t * r
    z = (jnp.dot(h_ref[...], w1t_ref[...], preferred_element_type=jnp.float32)
         + jnp.dot(agg, w2t_ref[...], preferred_element_type=jnp.float32)
         + bs_ref[...])
    h_next = _ln_relu(z, g_ref, be_ref)
    h_out[...] = h_next
    a = jnp.dot(h_next, wa2n_ref[...], preferred_element_type=jnp.float32)
    w = jnp.exp(a[:, 1:2])
    edn_ref[...] = jnp.exp(a[:, 0:1] + ban_ref[0, 0])
    padded = jnp.concatenate(
        [h_next * w, w,
         jnp.zeros((h_next.shape[0], g_next * w_next - 257), jnp.float32)],
        axis=1)
    for i, ref in enumerate(tbl_outs):
        ref[...] = padded[:, i * w_next:(i + 1) * w_next]


def _make_combine_prep(d, w_grp, g, g_next, w_next):
    return pl.pallas_call(
        functools.partial(_combine_prep_body, g, d, g_next, w_next),
        grid=(GRID,),
        in_specs=[pl.BlockSpec((BLK, w_grp), lambda i: (i, 0))] * g + [
            pl.BlockSpec((BLK, 1), lambda i: (i, 0)),
            pl.BlockSpec((BLK, d), lambda i: (i, 0)),
            pl.BlockSpec((d, 256), lambda i: (0, 0)),
            pl.BlockSpec((d, 256), lambda i: (0, 0)),
            pl.BlockSpec((1, 256), lambda i: (0, 0)),
            pl.BlockSpec((1, 256), lambda i: (0, 0)),
            pl.BlockSpec((1, 256), lambda i: (0, 0)),
            pl.BlockSpec((256, 2), lambda i: (0, 0)),
            pl.BlockSpec((1, 1), lambda i: (0, 0)),
        ],
        out_specs=[pl.BlockSpec((BLK, 256), lambda i: (i, 0))]
        + [pl.BlockSpec((BLK, w_next), lambda i: (i, 0))] * g_next
        + [pl.BlockSpec((BLK, 1), lambda i: (i, 0))],
        out_shape=[jax.ShapeDtypeStruct((N, 256), jnp.float32)]
        + [jax.ShapeDtypeStruct((N, w_next), jnp.float32)] * g_next
        + [jax.ShapeDtypeStruct((N, 1), jnp.float32)],
    )


def _combine_final_body(g, d, *refs):
    u_refs = refs[:g]
    (ed_ref, h_ref, w1t_ref, w2t_ref, bs_ref, g_ref, be_ref, x_ref,
     wskipt_ref, bskip_ref, acc_ref) = refs[g:]
    u = jnp.concatenate([r[...] for r in u_refs], axis=1)
    t = u[:, :d]
    s_sum = u[:, d:d + 1]
    ed = ed_ref[...]
    r = ed / jnp.maximum(ed * s_sum, 1e-8)
    agg = t * r
    z = (jnp.dot(h_ref[...], w1t_ref[...], preferred_element_type=jnp.float32)
         + jnp.dot(agg, w2t_ref[...], preferred_element_type=jnp.float32)
         + bs_ref[...])
    h3 = _ln_relu(z, g_ref, be_ref)
    xs = jnp.sum(x_ref[...], axis=0, keepdims=True)
    part = (jnp.sum(h3, axis=0, keepdims=True)
            + jnp.dot(xs, wskipt_ref[...], preferred_element_type=jnp.float32))

    @pl.when(pl.program_id(0) == 0)
    def _():
        acc_ref[...] = jnp.zeros_like(acc_ref)

    acc_ref[...] += part

    @pl.when(pl.program_id(0) == GRID - 1)
    def _():
        acc_ref[...] = acc_ref[...] * (1.0 / N) + bskip_ref[...]


def _make_combine_final(d, w_grp, g, in_dim):
    return pl.pallas_call(
        functools.partial(_combine_final_body, g, d),
        grid=(GRID,),
        in_specs=[pl.BlockSpec((BLK, w_grp), lambda i: (i, 0))] * g + [
            pl.BlockSpec((BLK, 1), lambda i: (i, 0)),
            pl.BlockSpec((BLK, d), lambda i: (i, 0)),
            pl.BlockSpec((d, 256), lambda i: (0, 0)),
            pl.BlockSpec((d, 256), lambda i: (0, 0)),
            pl.BlockSpec((1, 256), lambda i: (0, 0)),
            pl.BlockSpec((1, 256), lambda i: (0, 0)),
            pl.BlockSpec((1, 256), lambda i: (0, 0)),
            pl.BlockSpec((BLK, in_dim), lambda i: (i, 0)),
            pl.BlockSpec((in_dim, 256), lambda i: (0, 0)),
            pl.BlockSpec((1, 256), lambda i: (0, 0)),
        ],
        out_specs=pl.BlockSpec((1, 256), lambda i: (0, 0)),
        out_shape=jax.ShapeDtypeStruct((1, 256), jnp.float32),
    )


def kernel(x, edge_index, Ws0, bs0, g0, be0, Wa0, ba0, Ws1, bs1, g1, be1,
           Wa1, ba1, Ws2, bs2, g2, be2, Wa2, ba2, Wskip, bskip):
    pad = E_PAD - E
    src3 = jnp.concatenate(
        [edge_index[0], jnp.zeros((pad,), jnp.int32)]).reshape(
            NTILES, K_PT, NS, CH)
    dst3 = jnp.concatenate(
        [edge_index[1], jnp.full((pad,), N, jnp.int32)]).reshape(
            NTILES, K_PT, NS, CH)

    dims = (128, 256, 256)
    grouping = ((72, 2), (72, 4), (72, 4))
    params = ((Ws0, bs0, g0, be0, Wa0, ba0),
              (Ws1, bs1, g1, be1, Wa1, ba1),
              (Ws2, bs2, g2, be2, Wa2, ba2))

    wa2s = [jnp.stack([params[i][4][0, :dims[i]], params[i][4][0, dims[i]:]],
                      axis=1) for i in range(3)]
    bas = [params[i][5].reshape(1, 1) for i in range(3)]
    zblk = jnp.zeros((NPT, 72), jnp.float32)
    zblk8 = jnp.zeros((8, 72), jnp.float32)

    h = x
    tbls = None
    for i in range(3):
        d = dims[i]
        w_grp, g = grouping[i]
        Ws, bs, gg, be, Wa, ba = params[i]
        if i == 0:
            outs = _make_prep(d, w_grp, g)(h, wa2s[0], bas[0])
            tbls, ed = outs[:-1], outs[-1]
        us = _make_sc(w_grp, g)(*tbls, src3, dst3, zblk, zblk8)
        w1t = Ws[:, :d].T
        w2t = Ws[:, d:].T
        if i < 2:
            w_next, g_next = grouping[i + 1][0], grouping[i + 1][1]
            outs = _make_combine_prep(d, w_grp, g, g_next, w_next)(
                *us, ed, h, w1t, w2t,
                bs.reshape(1, 256), gg.reshape(1, 256), be.reshape(1, 256),
                wa2s[i + 1], bas[i + 1])
            h = outs[0]
            tbls = outs[1:1 + g_next]
            ed = outs[1 + g_next]
        else:
            acc = _make_combine_final(d, w_grp, g, 128)(
                *us, ed, h, w1t, w2t,
                bs.reshape(1, 256), gg.reshape(1, 256), be.reshape(1, 256),
                x, Wskip.T, bskip.reshape(1, 256))
    return acc.reshape(256)


# speedup vs baseline: 1.9115x; 1.0054x over previous
import functools

import jax
import jax.numpy as jnp
from jax import lax
from jax.experimental import pallas as pl
from jax.experimental.pallas import tpu as pltpu
from jax.experimental.pallas import tpu_sc as plsc

N = 10000
E = 320000
CH = 128
NS = 2
NTILES = 16
NPT = N // NTILES
K_PT = 80
E_PAD = NTILES * K_PT * NS * CH
BLK = 2000
GRID = N // BLK


def _prep_body(h_ref, wa_ref, ba_ref, *out_refs):
    ed_ref = out_refs[-1]
    tbl_refs = out_refs[:-1]
    h = h_ref[...]
    d = h.shape[1]
    w_grp = tbl_refs[0].shape[1]
    a = jnp.dot(h, wa_ref[...], preferred_element_type=jnp.float32)
    a_dst = a[:, 0:1]
    a_src = a[:, 1:2]
    w = jnp.exp(a_src)
    ed_ref[...] = jnp.exp(a_dst + ba_ref[0, 0])
    hw = h * w
    g = len(tbl_refs)
    pad = jnp.concatenate(
        [hw, w, jnp.zeros((h.shape[0], g * w_grp - d - 1), jnp.float32)],
        axis=1)
    for i, ref in enumerate(tbl_refs):
        ref[...] = pad[:, i * w_grp:(i + 1) * w_grp]


def _make_prep(d, w_grp, g):
    return pl.pallas_call(
        _prep_body,
        grid=(GRID,),
        in_specs=[
            pl.BlockSpec((BLK, d), lambda i: (i, 0)),
            pl.BlockSpec((d, 2), lambda i: (0, 0)),
            pl.BlockSpec((1, 1), lambda i: (0, 0)),
        ],
        out_specs=[pl.BlockSpec((BLK, w_grp), lambda i: (i, 0))] * g
        + [pl.BlockSpec((BLK, 1), lambda i: (i, 0))],
        out_shape=[jax.ShapeDtypeStruct((N, w_grp), jnp.float32)] * g
        + [jax.ShapeDtypeStruct((N, 1), jnp.float32)],
    )


def _make_sc(w_grp, g):
    mesh = plsc.VectorSubcoreMesh(core_axis_name="c", subcore_axis_name="s")
    g2 = g // 2

    @functools.partial(
        pl.kernel,
        mesh=mesh,
        compiler_params=pltpu.CompilerParams(use_tc_tiling_on_sc=False),
        out_type=[jax.ShapeDtypeStruct((N, w_grp), jnp.float32)] * g,
        scratch_types=[
            pltpu.VMEM((4, NS, CH), jnp.int32),
            pltpu.VMEM((4, NS, CH), jnp.int32),
            pltpu.VMEM((NS * CH, w_grp), jnp.float32),
            pltpu.VMEM((NS * CH, w_grp), jnp.float32),
            pltpu.VMEM_SHARED((N, w_grp), jnp.float32),
            pltpu.VMEM_SHARED((N + 8, w_grp), jnp.float32),
            pltpu.SemaphoreType.DMA,
            pltpu.SemaphoreType.DMA,
            pltpu.SemaphoreType.DMA,
            pltpu.SemaphoreType.DMA,
            pltpu.SemaphoreType.DMA,
            pltpu.SemaphoreType.DMA,
            pltpu.SemaphoreType.DMA,
            pltpu.SemaphoreType.DMA,
        ],
    )
    def sc_kernel(*refs):
        ins = refs[:g + 4]
        tbls = ins[:g]
        src_r, dst_r, zblk, zblk8 = ins[g:]
        outs = refs[g + 4:2 * g + 4]
        (sidx, didx, rows0, rows1, tbl, ush,
         si0, si1, si2, si3, sg0, sg1, ss0, ss1) = refs[2 * g + 4:]

        c = lax.axis_index("c")
        s = lax.axis_index("s")
        slab = pl.ds(s * NPT, NPT)

        rows = (rows0, rows1)
        si = (si0, si1, si2, si3)
        sg = (sg0, sg1)
        ss = (ss0, ss1)

        def fetch_idx(k, u):
            pltpu.async_copy(src_r.at[s, k], sidx.at[u], si[u])
            pltpu.async_copy(dst_r.at[s, k], didx.at[u], si[u])

        def wait_idx(k, u):
            pltpu.make_async_copy(src_r.at[s, k], sidx.at[u], si[u]).wait()
            pltpu.make_async_copy(dst_r.at[s, k], didx.at[u], si[u]).wait()

        def issue_gather(tbl_ref, u, buf, sem):
            for q in range(NS):
                pltpu.async_copy(tbl_ref.at[sidx.at[u, q]],
                                 buf.at[pl.ds(q * CH, CH)], sem)

        def wait_gather(tbl_ref, u, buf, sem):
            for q in range(NS):
                pltpu.make_async_copy(tbl_ref.at[sidx.at[u, q]],
                                      buf.at[pl.ds(q * CH, CH)], sem).wait()

        def issue_scatter(u, buf, sem):
            for q in range(NS):
                pltpu.async_copy(buf.at[pl.ds(q * CH, CH)],
                                 ush.at[didx.at[u, q]], sem, add=True)

        def wait_scatter(u, buf, sem):
            for q in range(NS):
                pltpu.make_async_copy(buf.at[pl.ds(q * CH, CH)],
                                      ush.at[didx.at[u, q]], sem).wait()

        def run_pass(tbl_hbm, out_hbm):
            pltpu.sync_copy(tbl_hbm.at[slab], tbl.at[slab])
            pltpu.sync_copy(zblk, ush.at[slab])

            @pl.when(s == NTILES - 1)
            def _():
                pltpu.sync_copy(zblk8, ush.at[pl.ds(N, 8)])

            plsc.subcore_barrier()

            fetch_idx(0, 0)
            fetch_idx(1, 1)
            wait_idx(0, 0)
            issue_gather(tbl, 0, rows[0], sg[0])

            def body_fn(j, carry):
                for r in range(4):
                    k = 4 * j + r
                    b = r % 2
                    nb = 1 - b
                    wait_gather(tbl, r, rows[b], sg[b])

                    @pl.when(k >= 1)
                    def _():
                        wait_scatter((r + 3) % 4, rows[nb], ss[nb])

                    @pl.when(k + 2 < K_PT)
                    def _():
                        fetch_idx(k + 2, (r + 2) % 4)

                    @pl.when(k + 1 < K_PT)
                    def _():
                        wait_idx(k + 1, (r + 1) % 4)
                        issue_gather(tbl, (r + 1) % 4, rows[nb], sg[nb])

                    issue_scatter(r, rows[b], ss[b])
                return carry

            lax.fori_loop(0, K_PT // 4, body_fn, 0)
            wait_scatter(3, rows[1], ss[1])

            plsc.subcore_barrier()
            pltpu.sync_copy(ush.at[slab], out_hbm.at[slab])

        @pl.when(c == 0)
        def _():
            for i in range(g2):
                run_pass(tbls[i], outs[i])

        @pl.when(c == 1)
        def _():
            for i in range(g2):
                run_pass(tbls[g2 + i], outs[g2 + i])

    return sc_kernel


def _ln_relu(z, g_ref, be_ref):
    mu = jnp.mean(z, axis=1, keepdims=True)
    var = jnp.mean((z - mu) ** 2, axis=1, keepdims=True)
    return jnp.maximum(
        (z - mu) * lax.rsqrt(var + 1e-5) * g_ref[...] + be_ref[...], 0.0)


def _combine_prep_body(g, d, g_next, w_next, *refs):
    u_refs = refs[:g]
    (ed_ref, h_ref, w1t_ref, w2t_ref, bs_ref, g_ref, be_ref,
     wa2n_ref, ban_ref) = refs[g:g + 9]
    h_out = refs[g + 9]
    tbl_outs = refs[g + 10:g + 10 + g_next]
    edn_ref = refs[g + 10 + g_next]
    u = jnp.concatenate([r[...] for r in u_refs], axis=1)
    t = u[:, :d]
    s_sum = u[:, d:d + 1]
    ed = ed_ref[...]
    r = ed / jnp.maximum(ed * s_sum, 1e-8)
    agg = t * r
    z = (jnp.dot(h_ref[...], w1t_ref[...], preferred_element_type=jnp.float32)
         + jnp.dot(agg, w2t_ref[...], preferred_element_type=jnp.float32)
         + bs_ref[...])
    h_next = _ln_relu(z, g_ref, be_ref)
    h_out[...] = h_next
    a = jnp.dot(h_next, wa2n_ref[...], preferred_element_type=jnp.float32)
    w = jnp.exp(a[:, 1:2])
    edn_ref[...] = jnp.exp(a[:, 0:1] + ban_ref[0, 0])
    padded = jnp.concatenate(
        [h_next * w, w,
         jnp.zeros((h_next.shape[0], g_next * w_next - 257), jnp.float32)],
        axis=1)
    for i, ref in enumerate(tbl_outs):
        ref[...] = padded[:, i * w_next:(i + 1) * w_next]


def _make_combine_prep(d, w_grp, g, g_next, w_next):
    return pl.pallas_call(
        functools.partial(_combine_prep_body, g, d, g_next, w_next),
        grid=(GRID,),
        in_specs=[pl.BlockSpec((BLK, w_grp), lambda i: (i, 0))] * g + [
            pl.BlockSpec((BLK, 1), lambda i: (i, 0)),
            pl.BlockSpec((BLK, d), lambda i: (i, 0)),
            pl.BlockSpec((d, 256), lambda i: (0, 0)),
            pl.BlockSpec((d, 256), lambda i: (0, 0)),
            pl.BlockSpec((1, 256), lambda i: (0, 0)),
            pl.BlockSpec((1, 256), lambda i: (0, 0)),
            pl.BlockSpec((1, 256), lambda i: (0, 0)),
            pl.BlockSpec((256, 2), lambda i: (0, 0)),
            pl.BlockSpec((1, 1), lambda i: (0, 0)),
        ],
        out_specs=[pl.BlockSpec((BLK, 256), lambda i: (i, 0))]
        + [pl.BlockSpec((BLK, w_next), lambda i: (i, 0))] * g_next
        + [pl.BlockSpec((BLK, 1), lambda i: (i, 0))],
        out_shape=[jax.ShapeDtypeStruct((N, 256), jnp.float32)]
        + [jax.ShapeDtypeStruct((N, w_next), jnp.float32)] * g_next
        + [jax.ShapeDtypeStruct((N, 1), jnp.float32)],
    )


def _combine_final_body(g, d, *refs):
    u_refs = refs[:g]
    (ed_ref, h_ref, w1t_ref, w2t_ref, bs_ref, g_ref, be_ref, x_ref,
     wskipt_ref, bskip_ref, acc_ref) = refs[g:]
    u = jnp.concatenate([r[...] for r in u_refs], axis=1)
    t = u[:, :d]
    s_sum = u[:, d:d + 1]
    ed = ed_ref[...]
    r = ed / jnp.maximum(ed * s_sum, 1e-8)
    agg = t * r
    z = (jnp.dot(h_ref[...], w1t_ref[...], preferred_element_type=jnp.float32)
         + jnp.dot(agg, w2t_ref[...], preferred_element_type=jnp.float32)
         + bs_ref[...])
    h3 = _ln_relu(z, g_ref, be_ref)
    xs = jnp.sum(x_ref[...], axis=0, keepdims=True)
    part = (jnp.sum(h3, axis=0, keepdims=True)
            + jnp.dot(xs, wskipt_ref[...], preferred_element_type=jnp.float32))

    @pl.when(pl.program_id(0) == 0)
    def _():
        acc_ref[...] = jnp.zeros_like(acc_ref)

    acc_ref[...] += part

    @pl.when(pl.program_id(0) == GRID - 1)
    def _():
        acc_ref[...] = acc_ref[...] * (1.0 / N) + bskip_ref[...]


def _make_combine_final(d, w_grp, g, in_dim):
    return pl.pallas_call(
        functools.partial(_combine_final_body, g, d),
        grid=(GRID,),
        in_specs=[pl.BlockSpec((BLK, w_grp), lambda i: (i, 0))] * g + [
            pl.BlockSpec((BLK, 1), lambda i: (i, 0)),
            pl.BlockSpec((BLK, d), lambda i: (i, 0)),
            pl.BlockSpec((d, 256), lambda i: (0, 0)),
            pl.BlockSpec((d, 256), lambda i: (0, 0)),
            pl.BlockSpec((1, 256), lambda i: (0, 0)),
            pl.BlockSpec((1, 256), lambda i: (0, 0)),
            pl.BlockSpec((1, 256), lambda i: (0, 0)),
            pl.BlockSpec((BLK, in_dim), lambda i: (i, 0)),
            pl.BlockSpec((in_dim, 256), lambda i: (0, 0)),
            pl.BlockSpec((1, 256), lambda i: (0, 0)),
        ],
        out_specs=pl.BlockSpec((1, 256), lambda i: (0, 0)),
        out_shape=jax.ShapeDtypeStruct((1, 256), jnp.float32),
    )


def kernel(x, edge_index, Ws0, bs0, g0, be0, Wa0, ba0, Ws1, bs1, g1, be1,
           Wa1, ba1, Ws2, bs2, g2, be2, Wa2, ba2, Wskip, bskip):
    pad = E_PAD - E
    src3 = jnp.concatenate(
        [edge_index[0], jnp.zeros((pad,), jnp.int32)]).reshape(
            NTILES, K_PT, NS, CH)
    dst3 = jnp.concatenate(
        [edge_index[1], jnp.full((pad,), N, jnp.int32)]).reshape(
            NTILES, K_PT, NS, CH)

    dims = (128, 256, 256)
    grouping = ((72, 2), (72, 4), (72, 4))
    params = ((Ws0, bs0, g0, be0, Wa0, ba0),
              (Ws1, bs1, g1, be1, Wa1, ba1),
              (Ws2, bs2, g2, be2, Wa2, ba2))

    wa2s = [jnp.stack([params[i][4][0, :dims[i]], params[i][4][0, dims[i]:]],
                      axis=1) for i in range(3)]
    bas = [params[i][5].reshape(1, 1) for i in range(3)]
    zblk = jnp.zeros((NPT, 72), jnp.float32)
    zblk8 = jnp.zeros((8, 72), jnp.float32)

    h = x
    tbls = None
    for i in range(3):
        d = dims[i]
        w_grp, g = grouping[i]
        Ws, bs, gg, be, Wa, ba = params[i]
        if i == 0:
            outs = _make_prep(d, w_grp, g)(h, wa2s[0], bas[0])
            tbls, ed = outs[:-1], outs[-1]
        us = _make_sc(w_grp, g)(*tbls, src3, dst3, zblk, zblk8)
        w1t = Ws[:, :d].T
        w2t = Ws[:, d:].T
        if i < 2:
            w_next, g_next = grouping[i + 1][0], grouping[i + 1][1]
            outs = _make_combine_prep(d, w_grp, g, g_next, w_next)(
                *us, ed, h, w1t, w2t,
                bs.reshape(1, 256), gg.reshape(1, 256), be.reshape(1, 256),
                wa2s[i + 1], bas[i + 1])
            h = outs[0]
            tbls = outs[1:1 + g_next]
            ed = outs[1 + g_next]
        else:
            acc = _make_combine_final(d, w_grp, g, 128)(
                *us, ed, h, w1t, w2t,
                bs.reshape(1, 256), gg.reshape(1, 256), be.reshape(1, 256),
                x, Wskip.T, bskip.reshape(1, 256))
    return acc.reshape(256)


# speedup vs baseline: 1.9227x; 1.0059x over previous
import functools

import jax
import jax.numpy as jnp
from jax import lax
from jax.experimental import pallas as pl
from jax.experimental.pallas import tpu as pltpu
from jax.experimental.pallas import tpu_sc as plsc

N = 10000
E = 320000
CH = 128
NS = 2
NTILES = 16
NPT = N // NTILES
K_PT = 80
E_PAD = NTILES * K_PT * NS * CH
BLK = 2000
GRID = N // BLK


def _prep_body(h_ref, wa_ref, ba_ref, *out_refs):
    ed_ref = out_refs[-1]
    tbl_refs = out_refs[:-1]
    h = h_ref[...]
    d = h.shape[1]
    w_grp = tbl_refs[0].shape[1]
    a = jnp.dot(h, wa_ref[...], preferred_element_type=jnp.float32)
    a_dst = a[:, 0:1]
    a_src = a[:, 1:2]
    w = jnp.exp(a_src)
    ed_ref[...] = jnp.exp(a_dst + ba_ref[0, 0])
    hw = h * w
    g = len(tbl_refs)
    pad = jnp.concatenate(
        [hw, w, jnp.zeros((h.shape[0], g * w_grp - d - 1), jnp.float32)],
        axis=1)
    for i, ref in enumerate(tbl_refs):
        ref[...] = pad[:, i * w_grp:(i + 1) * w_grp]


def _make_prep(d, w_grp, g):
    return pl.pallas_call(
        _prep_body,
        grid=(GRID,),
        in_specs=[
            pl.BlockSpec((BLK, d), lambda i: (i, 0)),
            pl.BlockSpec((d, 2), lambda i: (0, 0)),
            pl.BlockSpec((1, 1), lambda i: (0, 0)),
        ],
        out_specs=[pl.BlockSpec((BLK, w_grp), lambda i: (i, 0))] * g
        + [pl.BlockSpec((BLK, 1), lambda i: (i, 0))],
        out_shape=[jax.ShapeDtypeStruct((N, w_grp), jnp.float32)] * g
        + [jax.ShapeDtypeStruct((N, 1), jnp.float32)],
    )


def _make_sc(w_grp, g):
    mesh = plsc.VectorSubcoreMesh(core_axis_name="c", subcore_axis_name="s")
    g2 = g // 2

    @functools.partial(
        pl.kernel,
        mesh=mesh,
        compiler_params=pltpu.CompilerParams(use_tc_tiling_on_sc=False),
        out_type=[jax.ShapeDtypeStruct((N, w_grp), jnp.float32)] * g,
        scratch_types=[
            pltpu.VMEM((4, NS, CH), jnp.int32),
            pltpu.VMEM((4, NS, CH), jnp.int32),
            pltpu.VMEM((NS * CH, w_grp), jnp.float32),
            pltpu.VMEM((NS * CH, w_grp), jnp.float32),
            pltpu.VMEM_SHARED((N, w_grp), jnp.float32),
            pltpu.VMEM_SHARED((N + 8, w_grp), jnp.float32),
            pltpu.SemaphoreType.DMA,
            pltpu.SemaphoreType.DMA,
            pltpu.SemaphoreType.DMA,
            pltpu.SemaphoreType.DMA,
            pltpu.SemaphoreType.DMA,
            pltpu.SemaphoreType.DMA,
            pltpu.SemaphoreType.DMA,
            pltpu.SemaphoreType.DMA,
        ],
    )
    def sc_kernel(*refs):
        ins = refs[:g + 4]
        tbls = ins[:g]
        src_r, dst_r, zblk, zblk8 = ins[g:]
        outs = refs[g + 4:2 * g + 4]
        (sidx, didx, rows0, rows1, tbl, ush,
         si0, si1, si2, si3, sg0, sg1, ss0, ss1) = refs[2 * g + 4:]

        c = lax.axis_index("c")
        s = lax.axis_index("s")
        slab = pl.ds(s * NPT, NPT)

        rows = (rows0, rows1)
        si = (si0, si1, si2, si3)
        sg = (sg0, sg1)
        ss = (ss0, ss1)

        def fetch_idx(k, u):
            pltpu.async_copy(src_r.at[s, k], sidx.at[u], si[u])
            pltpu.async_copy(dst_r.at[s, k], didx.at[u], si[u])

        def wait_idx(k, u):
            pltpu.make_async_copy(src_r.at[s, k], sidx.at[u], si[u]).wait()
            pltpu.make_async_copy(dst_r.at[s, k], didx.at[u], si[u]).wait()

        def issue_gather(tbl_ref, u, buf, sem):
            for q in range(NS):
                pltpu.async_copy(tbl_ref.at[sidx.at[u, q]],
                                 buf.at[pl.ds(q * CH, CH)], sem)

        def wait_gather(tbl_ref, u, buf, sem):
            for q in range(NS):
                pltpu.make_async_copy(tbl_ref.at[sidx.at[u, q]],
                                      buf.at[pl.ds(q * CH, CH)], sem).wait()

        def issue_scatter(u, buf, sem):
            for q in range(NS):
                pltpu.async_copy(buf.at[pl.ds(q * CH, CH)],
                                 ush.at[didx.at[u, q]], sem, add=True)

        def wait_scatter(u, buf, sem):
            for q in range(NS):
                pltpu.make_async_copy(buf.at[pl.ds(q * CH, CH)],
                                      ush.at[didx.at[u, q]], sem).wait()

        def run_pass(tbl_hbm, out_hbm):
            pltpu.async_copy(tbl_hbm.at[slab], tbl.at[slab], sg[0])
            pltpu.async_copy(zblk, ush.at[slab], sg[1])
            fetch_idx(0, 0)
            fetch_idx(1, 1)

            @pl.when(s == NTILES - 1)
            def _():
                pltpu.sync_copy(zblk8, ush.at[pl.ds(N, 8)])

            pltpu.make_async_copy(tbl_hbm.at[slab], tbl.at[slab], sg[0]).wait()
            pltpu.make_async_copy(zblk, ush.at[slab], sg[1]).wait()
            plsc.subcore_barrier()

            wait_idx(0, 0)
            issue_gather(tbl, 0, rows[0], sg[0])

            def body_fn(j, carry):
                for r in range(4):
                    k = 4 * j + r
                    b = r % 2
                    nb = 1 - b
                    wait_gather(tbl, r, rows[b], sg[b])

                    @pl.when(k >= 1)
                    def _():
                        wait_scatter((r + 3) % 4, rows[nb], ss[nb])

                    @pl.when(k + 2 < K_PT)
                    def _():
                        fetch_idx(k + 2, (r + 2) % 4)

                    @pl.when(k + 1 < K_PT)
                    def _():
                        wait_idx(k + 1, (r + 1) % 4)
                        issue_gather(tbl, (r + 1) % 4, rows[nb], sg[nb])

                    issue_scatter(r, rows[b], ss[b])
                return carry

            lax.fori_loop(0, K_PT // 4, body_fn, 0)
            wait_scatter(3, rows[1], ss[1])

            plsc.subcore_barrier()
            pltpu.sync_copy(ush.at[slab], out_hbm.at[slab])

        @pl.when(c == 0)
        def _():
            for i in range(g2):
                run_pass(tbls[i], outs[i])

        @pl.when(c == 1)
        def _():
            for i in range(g2):
                run_pass(tbls[g2 + i], outs[g2 + i])

    return sc_kernel


def _ln_relu(z, g_ref, be_ref):
    mu = jnp.mean(z, axis=1, keepdims=True)
    var = jnp.mean((z - mu) ** 2, axis=1, keepdims=True)
    return jnp.maximum(
        (z - mu) * lax.rsqrt(var + 1e-5) * g_ref[...] + be_ref[...], 0.0)


def _combine_prep_body(g, d, g_next, w_next, *refs):
    u_refs = refs[:g]
    (ed_ref, h_ref, w1t_ref, w2t_ref, bs_ref, g_ref, be_ref,
     wa2n_ref, ban_ref) = refs[g:g + 9]
    h_out = refs[g + 9]
    tbl_outs = refs[g + 10:g + 10 + g_next]
    edn_ref = refs[g + 10 + g_next]
    u = jnp.concatenate([r[...] for r in u_refs], axis=1)
    t = u[:, :d]
    s_sum = u[:, d:d + 1]
    ed = ed_ref[...]
    r = ed / jnp.maximum(ed * s_sum, 1e-8)
    agg = t * r
    z = (jnp.dot(h_ref[...], w1t_ref[...], preferred_element_type=jnp.float32)
         + jnp.dot(agg, w2t_ref[...], preferred_element_type=jnp.float32)
         + bs_ref[...])
    h_next = _ln_relu(z, g_ref, be_ref)
    h_out[...] = h_next
    a = jnp.dot(h_next, wa2n_ref[...], preferred_element_type=jnp.float32)
    w = jnp.exp(a[:, 1:2])
    edn_ref[...] = jnp.exp(a[:, 0:1] + ban_ref[0, 0])
    padded = jnp.concatenate(
        [h_next * w, w,
         jnp.zeros((h_next.shape[0], g_next * w_next - 257), jnp.float32)],
        axis=1)
    for i, ref in enumerate(tbl_outs):
        ref[...] = padded[:, i * w_next:(i + 1) * w_next]


def _make_combine_prep(d, w_grp, g, g_next, w_next):
    return pl.pallas_call(
        functools.partial(_combine_prep_body, g, d, g_next, w_next),
        grid=(GRID,),
        in_specs=[pl.BlockSpec((BLK, w_grp), lambda i: (i, 0))] * g + [
            pl.BlockSpec((BLK, 1), lambda i: (i, 0)),
            pl.BlockSpec((BLK, d), lambda i: (i, 0)),
            pl.BlockSpec((d, 256), lambda i: (0, 0)),
            pl.BlockSpec((d, 256), lambda i: (0, 0)),
            pl.BlockSpec((1, 256), lambda i: (0, 0)),
            pl.BlockSpec((1, 256), lambda i: (0, 0)),
            pl.BlockSpec((1, 256), lambda i: (0, 0)),
            pl.BlockSpec((256, 2), lambda i: (0, 0)),
            pl.BlockSpec((1, 1), lambda i: (0, 0)),
        ],
        out_specs=[pl.BlockSpec((BLK, 256), lambda i: (i, 0))]
        + [pl.BlockSpec((BLK, w_next), lambda i: (i, 0))] * g_next
        + [pl.BlockSpec((BLK, 1), lambda i: (i, 0))],
        out_shape=[jax.ShapeDtypeStruct((N, 256), jnp.float32)]
        + [jax.ShapeDtypeStruct((N, w_next), jnp.float32)] * g_next
        + [jax.ShapeDtypeStruct((N, 1), jnp.float32)],
    )


def _combine_final_body(g, d, *refs):
    u_refs = refs[:g]
    (ed_ref, h_ref, w1t_ref, w2t_ref, bs_ref, g_ref, be_ref, x_ref,
     wskipt_ref, bskip_ref, acc_ref) = refs[g:]
    u = jnp.concatenate([r[...] for r in u_refs], axis=1)
    t = u[:, :d]
    s_sum = u[:, d:d + 1]
    ed = ed_ref[...]
    r = ed / jnp.maximum(ed * s_sum, 1e-8)
    agg = t * r
    z = (jnp.dot(h_ref[...], w1t_ref[...], preferred_element_type=jnp.float32)
         + jnp.dot(agg, w2t_ref[...], preferred_element_type=jnp.float32)
         + bs_ref[...])
    h3 = _ln_relu(z, g_ref, be_ref)
    xs = jnp.sum(x_ref[...], axis=0, keepdims=True)
    part = (jnp.sum(h3, axis=0, keepdims=True)
            + jnp.dot(xs, wskipt_ref[...], preferred_element_type=jnp.float32))

    @pl.when(pl.program_id(0) == 0)
    def _():
        acc_ref[...] = jnp.zeros_like(acc_ref)

    acc_ref[...] += part

    @pl.when(pl.program_id(0) == GRID - 1)
    def _():
        acc_ref[...] = acc_ref[...] * (1.0 / N) + bskip_ref[...]


def _make_combine_final(d, w_grp, g, in_dim):
    return pl.pallas_call(
        functools.partial(_combine_final_body, g, d),
        grid=(GRID,),
        in_specs=[pl.BlockSpec((BLK, w_grp), lambda i: (i, 0))] * g + [
            pl.BlockSpec((BLK, 1), lambda i: (i, 0)),
            pl.BlockSpec((BLK, d), lambda i: (i, 0)),
            pl.BlockSpec((d, 256), lambda i: (0, 0)),
            pl.BlockSpec((d, 256), lambda i: (0, 0)),
            pl.BlockSpec((1, 256), lambda i: (0, 0)),
            pl.BlockSpec((1, 256), lambda i: (0, 0)),
            pl.BlockSpec((1, 256), lambda i: (0, 0)),
            pl.BlockSpec((BLK, in_dim), lambda i: (i, 0)),
            pl.BlockSpec((in_dim, 256), lambda i: (0, 0)),
            pl.BlockSpec((1, 256), lambda i: (0, 0)),
        ],
        out_specs=pl.BlockSpec((1, 256), lambda i: (0, 0)),
        out_shape=jax.ShapeDtypeStruct((1, 256), jnp.float32),
    )


def kernel(x, edge_index, Ws0, bs0, g0, be0, Wa0, ba0, Ws1, bs1, g1, be1,
           Wa1, ba1, Ws2, bs2, g2, be2, Wa2, ba2, Wskip, bskip):
    pad = E_PAD - E
    src3 = jnp.concatenate(
        [edge_index[0], jnp.zeros((pad,), jnp.int32)]).reshape(
            NTILES, K_PT, NS, CH)
    dst3 = jnp.concatenate(
        [edge_index[1], jnp.full((pad,), N, jnp.int32)]).reshape(
            NTILES, K_PT, NS, CH)

    dims = (128, 256, 256)
    grouping = ((72, 2), (72, 4), (72, 4))
    params = ((Ws0, bs0, g0, be0, Wa0, ba0),
              (Ws1, bs1, g1, be1, Wa1, ba1),
              (Ws2, bs2, g2, be2, Wa2, ba2))

    wa2s = [jnp.stack([params[i][4][0, :dims[i]], params[i][4][0, dims[i]:]],
                      axis=1) for i in range(3)]
    bas = [params[i][5].reshape(1, 1) for i in range(3)]
    zblk = jnp.zeros((NPT, 72), jnp.float32)
    zblk8 = jnp.zeros((8, 72), jnp.float32)

    h = x
    tbls = None
    for i in range(3):
        d = dims[i]
        w_grp, g = grouping[i]
        Ws, bs, gg, be, Wa, ba = params[i]
        if i == 0:
            outs = _make_prep(d, w_grp, g)(h, wa2s[0], bas[0])
            tbls, ed = outs[:-1], outs[-1]
        us = _make_sc(w_grp, g)(*tbls, src3, dst3, zblk, zblk8)
        w1t = Ws[:, :d].T
        w2t = Ws[:, d:].T
        if i < 2:
            w_next, g_next = grouping[i + 1][0], grouping[i + 1][1]
            outs = _make_combine_prep(d, w_grp, g, g_next, w_next)(
                *us, ed, h, w1t, w2t,
                bs.reshape(1, 256), gg.reshape(1, 256), be.reshape(1, 256),
                wa2s[i + 1], bas[i + 1])
            h = outs[0]
            tbls = outs[1:1 + g_next]
            ed = outs[1 + g_next]
        else:
            acc = _make_combine_final(d, w_grp, g, 128)(
                *us, ed, h, w1t, w2t,
                bs.reshape(1, 256), gg.reshape(1, 256), be.reshape(1, 256),
                x, Wskip.T, bskip.reshape(1, 256))
    return acc.reshape(256)
